# baseline (speedup 1.0000x reference)
import functools

import jax
import jax.numpy as jnp
from jax import lax
from jax.experimental import pallas as pl
from jax.experimental.pallas import tpu as pltpu
from jax.experimental.pallas import tpu_sc as plsc

_B = 1024
_D = 128
_H = 100000
_DOUT = 128
_R = 64
_HB = 2048
_NBLK = 784
_HPAD = _NBLK * 128
_NW = 32
_RPW = _B // _NW


def _scores_body(x_ref, w_ref, out_ref, bm_ref):
    j = pl.program_id(0)
    s = lax.dot_general(
        x_ref[...], w_ref[...],
        dimension_numbers=(((1,), (1,)), ((), ())),
        preferred_element_type=jnp.float32,
    )
    col = j * _HB + lax.broadcasted_iota(jnp.int32, s.shape, 1)
    s = jnp.where(col < _H, s, -1e30)
    out_ref[...] = s
    cols = []
    for k in range(_HB // 128):
        cols.append(jnp.max(s[:, k * 128:(k + 1) * 128], axis=1, keepdims=True))
    bm_ref[...] = jnp.concatenate(cols, axis=1)[None]


def _scores(x, w_in):
    grid = (_HPAD // _HB,)
    return pl.pallas_call(
        _scores_body,
        grid=grid,
        in_specs=[
            pl.BlockSpec((_B, _D), lambda j: (0, 0)),
            pl.BlockSpec((_HB, _D), lambda j: (j, 0)),
        ],
        out_specs=[
            pl.BlockSpec((_B, _HB), lambda j: (0, j)),
            pl.BlockSpec((1, _B, _HB // 128), lambda j: (j, 0, 0)),
        ],
        out_shape=[
            jax.ShapeDtypeStruct((_B, _HPAD), jnp.float32),
            jax.ShapeDtypeStruct((_HPAD // _HB, _B, _HB // 128), jnp.float32),
        ],
    )(x, w_in)


def _sc_select_combine(x, bm, chunks, w_gate, w_out):
    mesh = plsc.VectorSubcoreMesh(core_axis_name="c", subcore_axis_name="s")

    @functools.partial(
        pl.kernel,
        mesh=mesh,
        out_type=jax.ShapeDtypeStruct((_B, _DOUT), jnp.float32),
        scratch_types=[
            pltpu.VMEM((_D,), jnp.float32),
            pltpu.VMEM((_NBLK,), jnp.float32),
            pltpu.VMEM((_NBLK,), jnp.uint32),
            pltpu.VMEM((96,), jnp.int32),
            pltpu.VMEM((64,), jnp.int32),
            pltpu.VMEM((64, 128), jnp.float32),
            pltpu.VMEM((8208,), jnp.uint32),
            pltpu.VMEM((8208,), jnp.int32),
            pltpu.VMEM((96,), jnp.uint32),
            pltpu.VMEM((96,), jnp.int32),
            pltpu.VMEM((64,), jnp.float32),
            pltpu.VMEM((64, 128), jnp.float32),
            pltpu.VMEM((64, 128), jnp.float32),
            pltpu.VMEM((_DOUT,), jnp.float32),
            pltpu.SemaphoreType.DMA,
        ],
        compiler_params=pltpu.CompilerParams(needs_layout_passes=False),
    )
    def k(x_hbm, bm_hbm, ch_hbm, wg_hbm, wo_hbm, out_hbm,
          xrow, bmrow, ukeys, blksel, blkg, cand, sk, scol, fk, fcol,
          coef, grows, vrows, orow, sem):
        wid = lax.axis_index("s") * 2 + lax.axis_index("c")
        iota = lax.broadcasted_iota(jnp.int32, (16,), 0)
        i32max = jnp.int32(0x7FFFFFFF)

        def ukey_of(v):
            i = plsc.bitcast(v, jnp.int32)
            s = i ^ (lax.shift_right_arithmetic(i, 31) & i32max)
            return plsc.bitcast(s, jnp.uint32) ^ jnp.uint32(0x80000000)

        def fval_of(u):
            s = plsc.bitcast(u ^ jnp.uint32(0x80000000), jnp.int32)
            i = s ^ (lax.shift_right_arithmetic(s, 31) & i32max)
            return plsc.bitcast(i, jnp.float32)

        def count_ge(buf, n16, t):
            tv = jnp.full((16,), t, jnp.uint32)

            def cnt(j, acc):
                return acc + (buf[pl.ds(j * 16, 16)] >= tv).astype(jnp.int32)

            accv = lax.fori_loop(0, n16, cnt, jnp.zeros((16,), jnp.int32))
            return jnp.sum(accv)

        def select64(buf, n16, k_want):
            def bit_step(t, T):
                bit = lax.shift_left(jnp.uint32(1), (31 - t).astype(jnp.uint32))
                cand_t = T | bit
                c = count_ge(buf, n16, cand_t)
                return jnp.where(c >= k_want, cand_t, T)

            return lax.fori_loop(0, 32, bit_step, jnp.uint32(0))

        def fill64(src, n16, T, dst_k, dst_i, idx_of):
            def mk_pass(strict):
                def step(j, cnt):
                    u = src[pl.ds(j * 16, 16)]
                    m = (u > T) if strict else (u == T)

                    @pl.when(cnt < 64)
                    def _():
                        plsc.store_compressed(dst_k.at[pl.ds(cnt, 16)], u, mask=m)
                        plsc.store_compressed(dst_i.at[pl.ds(cnt, 16)],
                                              idx_of(j), mask=m)

                    n = jnp.max(plsc.all_reduce_population_count(m))
                    return cnt + jnp.where(cnt < 64, n, 0)

                return step

            cnt = lax.fori_loop(0, n16, mk_pass(True), jnp.int32(0))
            cnt = lax.fori_loop(0, n16, mk_pass(False), cnt)
            return cnt

        c_gelu1 = jnp.float32(0.7978845608028654)
        c_gelu2 = jnp.float32(0.044715)

        def row_body(rb, _):
            b = wid * _RPW + rb
            pltpu.sync_copy(x_hbm.at[b], xrow)
            pltpu.sync_copy(bm_hbm.at[b], bmrow)

            def mkkey(j, _c):
                ukeys[pl.ds(j * 16, 16)] = ukey_of(bmrow[pl.ds(j * 16, 16)])
                return _c

            lax.fori_loop(0, _NBLK // 16, mkkey, jnp.int32(0))

            T1 = select64(ukeys, _NBLK // 16, 64)
            fill64(ukeys, _NBLK // 16, T1, fk, blksel,
                   lambda j: j * 16 + iota)

            def mkg(j, _c):
                blkg[pl.ds(j * 16, 16)] = (blksel[pl.ds(j * 16, 16)]
                                           + b * _NBLK)
                return _c

            lax.fori_loop(0, 4, mkg, jnp.int32(0))
            pltpu.async_copy(ch_hbm.at[blkg], cand, sem).wait()

            def scan_r(r, m):
                bg = plsc.load_gather(blksel, [jnp.full((16,), r, jnp.int32)])
                for c in range(8):
                    u = ukey_of(cand[r, pl.ds(c * 16, 16)])
                    msk = u >= T1
                    col = bg * 128 + (c * 16 + iota)
                    plsc.store_compressed(sk.at[pl.ds(m, 16)], u, mask=msk)
                    plsc.store_compressed(scol.at[pl.ds(m, 16)], col, mask=msk)
                    m = m + jnp.max(plsc.all_reduce_population_count(msk))
                return m

            M = lax.fori_loop(0, 64, scan_r, jnp.int32(0))
            sk[pl.ds(M, 16)] = jnp.zeros((16,), jnp.uint32)

            nv2 = (M + 15) // 16
            T2 = select64(sk, nv2, 64)
            fill64(sk, nv2, T2, fk, fcol,
                   lambda j: scol[pl.ds(j * 16, 16)])

            pltpu.async_copy(wg_hbm.at[fcol.at[pl.ds(0, 64)]], grows,
                             sem).wait()
            pltpu.async_copy(wo_hbm.at[fcol.at[pl.ds(0, 64)]], vrows,
                             sem).wait()

            def dot_r(r, _c):
                acc = jnp.zeros((16,), jnp.float32)
                for c in range(8):
                    acc = acc + (xrow[pl.ds(c * 16, 16)]
                                 * grows[r, pl.ds(c * 16, 16)])
                g = jnp.sum(acc)
                plsc.store_scatter(coef, [jnp.full((16,), r, jnp.int32)],
                                   jnp.full((16,), g, jnp.float32),
                                   mask=(iota == 0))
                return _c

            lax.fori_loop(0, 64, dot_r, jnp.int32(0))

            for j in range(4):
                g = coef[pl.ds(j * 16, 16)]
                sval = fval_of(fk[pl.ds(j * 16, 16)])
                t = c_gelu1 * (g + c_gelu2 * g * g * g)
                e = jnp.exp(jnp.float32(2.0) * t)
                tanh_t = jnp.float32(1.0) - jnp.float32(2.0) / (e + 1.0)
                gelu = jnp.float32(0.5) * g * (jnp.float32(1.0) + tanh_t)
                coef[pl.ds(j * 16, 16)] = sval * gelu

            def acc_r(r, carry):
                s = plsc.load_gather(coef, [jnp.full((16,), r, jnp.int32)])
                return tuple(carry[c] + s * vrows[r, pl.ds(c * 16, 16)]
                             for c in range(8))

            accs = lax.fori_loop(
                0, 64, acc_r,
                tuple(jnp.zeros((16,), jnp.float32) for _ in range(8)))
            for c in range(8):
                orow[pl.ds(c * 16, 16)] = accs[c]
            pltpu.sync_copy(orow, out_hbm.at[b])
            return jnp.int32(0)

        lax.fori_loop(0, _RPW, row_body, jnp.int32(0))

    return k(x, bm, chunks, w_gate, w_out)


def kernel(x_b_D, W_in, W_gate, W_out):
    x = x_b_D.reshape(-1, x_b_D.shape[-1])
    scores, bm3 = _scores(x, W_in)
    bm = bm3.transpose(1, 0, 2).reshape(_B, _NBLK)
    chunks = scores.reshape(_B * _NBLK, 128)
    out = _sc_select_combine(x, bm, chunks, W_gate, W_out)
    return out.reshape(x_b_D.shape[:-1] + (_DOUT,))
